# baseline (speedup 1.0000x reference)
import functools

import jax
import jax.numpy as jnp
from jax import lax
from jax.experimental import pallas as pl
from jax.experimental.pallas import tpu as pltpu
from jax.experimental.pallas import tpu_sc as plsc

N = 10000
D = 128
E = 320000

NC = 2
NS = 16
NW = NC * NS

NP = 10240
RPT = NP // NS

B = 128
NBT = 80
EP = NW * NBT * B
NBUF = 2
NBTA = 80

_MESH = dict(core_axis_name="c", subcore_axis_name="s", num_cores=NC,
             num_subcores=NS)


def _sc_deg(dst3, z1):
    @functools.partial(
        pl.kernel,
        out_type=jax.ShapeDtypeStruct((NC, NP), jnp.float32),
        mesh=plsc.VectorSubcoreMesh(**_MESH),
        scratch_types=[
            pltpu.VMEM((NBT, B), jnp.int32),
            pltpu.VMEM((B,), jnp.float32),
            pltpu.VMEM_SHARED((NP,), jnp.float32),
            pltpu.SemaphoreType.DMA,
        ],
    )
    def k(dst3_hbm, z1_hbm, degp_hbm, dstv, ones_v, dacc, sem):
        c = lax.axis_index("c")
        s = lax.axis_index("s")
        w = s * NC + c
        pltpu.sync_copy(z1_hbm.at[pl.ds(s * RPT, RPT)],
                        dacc.at[pl.ds(s * RPT, RPT)])
        for i in range(B // 16):
            ones_v[pl.ds(i * 16, 16)] = jnp.ones((16,), jnp.float32)
        pltpu.sync_copy(dst3_hbm.at[w], dstv)
        plsc.subcore_barrier()

        def body(j, carry):
            pltpu.async_copy(ones_v, dacc.at[dstv.at[j]], sem, add=True)
            return carry

        lax.fori_loop(0, NBT, body, 0)

        def drain(j, carry):
            pltpu.make_async_copy(ones_v, dacc.at[dstv.at[j]], sem).wait()
            return carry

        lax.fori_loop(0, NBT, drain, 0)
        plsc.subcore_barrier()
        pltpu.sync_copy(dacc.at[pl.ds(s * RPT, RPT)],
                        degp_hbm.at[c, pl.ds(s * RPT, RPT)])

    return k(dst3, z1)


def _make_sc_agg(do_t):
    NIB = 4

    outs = [jax.ShapeDtypeStruct((NP, D), jnp.float32)]
    scratch = [
        pltpu.VMEM((NIB, B), jnp.int32),
        pltpu.VMEM((NIB, B), jnp.int32),
        pltpu.VMEM((NBUF, B, D), jnp.float32),
        pltpu.VMEM_SHARED((NP, D), jnp.float32),
        pltpu.SemaphoreType.DMA((NBUF,)),
        pltpu.SemaphoreType.DMA((NIB,)),
        pltpu.SemaphoreType.DMA((NIB,)),
    ]
    if do_t:
        outs.append(jax.ShapeDtypeStruct((NP,), jnp.float32))
        scratch += [
            pltpu.VMEM((NBUF, B), jnp.float32),
            pltpu.VMEM((NP,), jnp.float32),
            pltpu.VMEM_SHARED((NP,), jnp.float32),
        ]

    @functools.partial(
        pl.kernel,
        out_type=tuple(outs) if do_t else outs[0],
        mesh=plsc.VectorSubcoreMesh(**_MESH),
        compiler_params=pltpu.CompilerParams(needs_layout_passes=False),
        scratch_types=scratch,
    )
    def k(hs_hbm, srcf_hbm, dstf_hbm, dis_hbm, z2_hbm, z1_hbm, *rest):
        if do_t:
            (agg_hbm, t_hbm, srcr, dstr, rows, acc, gsem, issem, idsem,
             val, disl, tacc) = rest
        else:
            agg_hbm, srcr, dstr, rows, acc, gsem, issem, idsem = rest
        c = lax.axis_index("c")
        s = lax.axis_index("s")
        base = s * (NBTA * B)

        def start_idx(j, q):
            pltpu.async_copy(srcf_hbm.at[pl.ds(base + j * B, B)],
                             srcr.at[q], issem.at[q])
            pltpu.async_copy(dstf_hbm.at[pl.ds(base + j * B, B)],
                             dstr.at[q], idsem.at[q])

        def wait_idx(j, q):
            pltpu.make_async_copy(srcf_hbm.at[pl.ds(base + j * B, B)],
                                  srcr.at[q], issem.at[q]).wait()
            pltpu.make_async_copy(dstf_hbm.at[pl.ds(base + j * B, B)],
                                  dstr.at[q], idsem.at[q]).wait()

        def pipeline(start_work, wait_work, finish_work):
            for q in range(NIB):
                start_idx(q, q)
            for u in range(NBUF):
                wait_idx(u, u)
                start_work(u, u)

            def group(g, carry):
                jb = g * NIB
                for u in range(NIB):
                    b = u % NBUF
                    q2 = (u + NBUF) % NIB
                    wait_work(b, u)
                    finish_work(b, u)
                    start_idx(jb + u + NIB, u)
                    wait_idx(jb + u + NBUF, q2)
                    start_work(b, q2)
                return carry

            lax.fori_loop(0, NBTA // NIB - 1, group, 0)
            tb = NBTA - NIB
            for u in range(NIB):
                b = u % NBUF
                q2 = (u + NBUF) % NIB
                wait_work(b, u)
                finish_work(b, u)
                if u < NIB - NBUF:
                    wait_idx(tb + u + NBUF, q2)
                    start_work(b, q2)

        @pl.when(c == 0)
        def _():
            pltpu.sync_copy(z2_hbm.at[pl.ds(s * RPT, RPT)],
                            acc.at[pl.ds(s * RPT, RPT)])
            plsc.subcore_barrier()
            pipeline(
                lambda b, q: pltpu.async_copy(hs_hbm.at[srcr.at[q]],
                                              rows.at[b], gsem.at[b]),
                lambda b, q: pltpu.make_async_copy(hs_hbm.at[srcr.at[q]],
                                                   rows.at[b],
                                                   gsem.at[b]).wait(),
                lambda b, q: pltpu.sync_copy(rows.at[b],
                                             acc.at[dstr.at[q]], add=True),
            )
            plsc.subcore_barrier()
            pltpu.sync_copy(acc.at[pl.ds(s * RPT, RPT)],
                            agg_hbm.at[pl.ds(s * RPT, RPT)])

        if do_t:
            def compute_val(b, q):
                for i in range(B // 16):
                    idxv = dstr[q, pl.ds(i * 16, 16)]
                    val[b, pl.ds(i * 16, 16)] = plsc.load_gather(
                        disl, [idxv])

            @pl.when(c == 1)
            def _():
                pltpu.sync_copy(z1_hbm.at[pl.ds(s * RPT, RPT)],
                                tacc.at[pl.ds(s * RPT, RPT)])
                pltpu.sync_copy(dis_hbm, disl)
                plsc.subcore_barrier()
                pipeline(
                    compute_val,
                    lambda b, q: None,
                    lambda b, q: pltpu.sync_copy(val.at[b],
                                                 tacc.at[srcr.at[q]],
                                                 add=True),
                )
                plsc.subcore_barrier()
                pltpu.sync_copy(tacc.at[pl.ds(s * RPT, RPT)],
                                t_hbm.at[pl.ds(s * RPT, RPT)])

    return k


_sc_agg_t = _make_sc_agg(True)
_sc_agg = _make_sc_agg(False)


RB = 640
GRID = NP // RB


def _tc1_body(x_ref, w1_ref, degp_ref, hs_ref, dis_ref):
    pid = pl.program_id(0)
    deg = degp_ref[0] + degp_ref[1] + 1.0
    rows = pid * RB + lax.broadcasted_iota(jnp.int32, (RB, 1), 0)
    dis = jnp.where(rows < N, lax.rsqrt(deg), 0.0)
    h = jnp.dot(x_ref[...], w1_ref[...], preferred_element_type=jnp.float32)
    hs_ref[...] = dis * h
    dis_ref[...] = dis


def _tc1(x_pad, w1, degp):
    return pl.pallas_call(
        _tc1_body,
        grid=(GRID,),
        in_specs=[
            pl.BlockSpec((RB, D), lambda i: (i, 0)),
            pl.BlockSpec((D, D), lambda i: (0, 0)),
            pl.BlockSpec((NC, RB, 1), lambda i: (0, i, 0)),
        ],
        out_specs=[
            pl.BlockSpec((RB, D), lambda i: (i, 0)),
            pl.BlockSpec((RB, 1), lambda i: (i, 0)),
        ],
        out_shape=[
            jax.ShapeDtypeStruct((NP, D), jnp.float32),
            jax.ShapeDtypeStruct((NP, 1), jnp.float32),
        ],
    )(x_pad, w1, degp)


def _tc2_body(aggp_ref, hs1_ref, dis_ref, b1_ref, w2_ref, hs2_ref):
    dis = dis_ref[...]
    a = aggp_ref[...] + hs1_ref[...]
    h1 = jnp.maximum(dis * a + b1_ref[...], 0.0)
    hs2_ref[...] = dis * jnp.dot(h1, w2_ref[...],
                                 preferred_element_type=jnp.float32)


def _tc2(aggp, hs1, dis, b1, w2):
    return pl.pallas_call(
        _tc2_body,
        grid=(GRID,),
        in_specs=[
            pl.BlockSpec((RB, D), lambda i: (i, 0)),
            pl.BlockSpec((RB, D), lambda i: (i, 0)),
            pl.BlockSpec((RB, 1), lambda i: (i, 0)),
            pl.BlockSpec((1, D), lambda i: (0, 0)),
            pl.BlockSpec((D, D), lambda i: (0, 0)),
        ],
        out_specs=pl.BlockSpec((RB, D), lambda i: (i, 0)),
        out_shape=jax.ShapeDtypeStruct((NP, D), jnp.float32),
    )(aggp, hs1, dis, b1, w2)


def _tc3_body(aggp_ref, hs2_ref, dis_ref, tp_ref, b2_ref, w3_ref, b3_ref,
              m1_ref, mb1_ref, m2_ref, mb2_ref, out_ref, zacc):
    pid = pl.program_id(0)
    dis = dis_ref[...]
    a = aggp_ref[...] + hs2_ref[...]
    h2 = jnp.maximum(dis * a + b2_ref[...], 0.0)
    cvec = dis * (tp_ref[...] + dis)
    part = jnp.sum(cvec * h2, axis=0, keepdims=True)

    @pl.when(pid == 0)
    def _():
        zacc[...] = jnp.zeros_like(zacc)

    zacc[...] += part

    @pl.when(pid == GRID - 1)
    def _():
        z = zacc[...] * (1.0 / N)
        g = jnp.dot(z, w3_ref[...], preferred_element_type=jnp.float32)
        g = g + b3_ref[...]
        g = jnp.maximum(
            jnp.dot(g, m1_ref[...], preferred_element_type=jnp.float32)
            + mb1_ref[...], 0.0)
        g = jnp.dot(g, m2_ref[...], preferred_element_type=jnp.float32)
        g = g + mb2_ref[...]
        out_ref[...] = g


def _tc3(aggp, hs2, dis, tp, b2, w3, b3, m1, mb1, m2, mb2):
    vec = pl.BlockSpec((1, D), lambda i: (0, 0))
    mat = pl.BlockSpec((D, D), lambda i: (0, 0))
    return pl.pallas_call(
        _tc3_body,
        grid=(GRID,),
        in_specs=[
            pl.BlockSpec((RB, D), lambda i: (i, 0)),
            pl.BlockSpec((RB, D), lambda i: (i, 0)),
            pl.BlockSpec((RB, 1), lambda i: (i, 0)),
            pl.BlockSpec((RB, 1), lambda i: (i, 0)),
            vec, mat, vec, mat, vec, mat, vec,
        ],
        out_specs=pl.BlockSpec((1, D), lambda i: (0, 0)),
        out_shape=jax.ShapeDtypeStruct((1, D), jnp.float32),
        scratch_shapes=[pltpu.VMEM((1, D), jnp.float32)],
    )(aggp, hs2, dis, tp, b2, w3, b3, m1, mb1, m2, mb2)


def kernel(x, edge_index, W1, b1, W2, b2, W3, b3, M1, mb1, M2, mb2):
    x_pad = jnp.pad(x, ((0, NP - N), (0, 0)))
    pad = jnp.full((EP - E,), NP - 1, dtype=jnp.int32)
    src_f = jnp.concatenate([edge_index[0], pad])
    dst_f = jnp.concatenate([edge_index[1], pad])
    dst3 = dst_f.reshape(NW, NBT, B)
    z1 = jnp.zeros((NP,), jnp.float32)
    z2 = jnp.zeros((NP, D), jnp.float32)

    EPH = NS * NBTA * B
    sA, sB = src_f[:EPH], src_f[EPH:]
    dA, dB = dst_f[:EPH], dst_f[EPH:]

    degp = _sc_deg(dst3, z1)
    hs1, dis = _tc1(x_pad, W1, degp.reshape(NC, NP, 1))
    dis1 = dis.reshape(NP)
    agg1a, ta = _sc_agg_t(hs1, sA, dA, dis1, z2, z1)
    agg1, t = _sc_agg_t(hs1, sB, dB, dis1, agg1a, ta)
    hs2 = _tc2(agg1, hs1, dis, b1.reshape(1, D), W2)
    agg2a = _sc_agg(hs2, sA, dA, dis1, z2, z1)
    agg2 = _sc_agg(hs2, sB, dB, dis1, agg2a, z1)
    g = _tc3(agg2, hs2, dis, t.reshape(NP, 1), b2.reshape(1, D),
             W3, b3.reshape(1, D), M1, mb1.reshape(1, D), M2,
             mb2.reshape(1, D))
    return g


# speedup vs baseline: 1.4715x; 1.4715x over previous
import functools

import jax
import jax.numpy as jnp
from jax import lax
from jax.experimental import pallas as pl
from jax.experimental.pallas import tpu as pltpu
from jax.experimental.pallas import tpu_sc as plsc

N = 10000
D = 128
E = 320000

NC = 2
NS = 16
NW = NC * NS

NP = 10240
RPT = NP // NS

B = 128
NBT = 79
EPW = NBT * B
EP = NW * EPW

_MESH = dict(core_axis_name="c", subcore_axis_name="s", num_cores=NC,
             num_subcores=NS)


def _sc_deg(dst3, z1):
    @functools.partial(
        pl.kernel,
        out_type=jax.ShapeDtypeStruct((NC, NP), jnp.float32),
        mesh=plsc.VectorSubcoreMesh(**_MESH),
        scratch_types=[
            pltpu.VMEM((NBT, B), jnp.int32),
            pltpu.VMEM((B,), jnp.float32),
            pltpu.VMEM_SHARED((NP,), jnp.float32),
        ],
    )
    def k(dst3_hbm, z1_hbm, degp_hbm, dstv, ones_v, dacc):
        c = lax.axis_index("c")
        s = lax.axis_index("s")
        w = s * NC + c
        pltpu.sync_copy(z1_hbm.at[pl.ds(s * RPT, RPT)],
                        dacc.at[pl.ds(s * RPT, RPT)])
        for i in range(B // 16):
            ones_v[pl.ds(i * 16, 16)] = jnp.ones((16,), jnp.float32)
        pltpu.sync_copy(dst3_hbm.at[w], dstv)
        plsc.subcore_barrier()

        def body(j, carry):
            pltpu.sync_copy(ones_v, dacc.at[dstv.at[j]], add=True)
            return carry

        lax.fori_loop(0, NBT, body, 0)
        plsc.subcore_barrier()
        pltpu.sync_copy(dacc.at[pl.ds(s * RPT, RPT)],
                        degp_hbm.at[c, pl.ds(s * RPT, RPT)])

    return k(dst3, z1)


def _make_sc_agg(do_t):
    outs = [jax.ShapeDtypeStruct((NC, NP, D), jnp.float32)]
    scratch = [
        pltpu.VMEM((NBT, B), jnp.int32),
        pltpu.VMEM((NBT, B), jnp.int32),
        pltpu.VMEM((B, D), jnp.float32),
        pltpu.VMEM_SHARED((NP, D), jnp.float32),
        pltpu.SemaphoreType.DMA,
    ]
    if do_t:
        outs.append(jax.ShapeDtypeStruct((NC, NP), jnp.float32))
        scratch += [
            pltpu.VMEM((B,), jnp.float32),
            pltpu.VMEM_SHARED((NP,), jnp.float32),
            pltpu.SemaphoreType.DMA,
        ]

    @functools.partial(
        pl.kernel,
        out_type=tuple(outs) if do_t else outs[0],
        mesh=plsc.VectorSubcoreMesh(**_MESH),
        scratch_types=scratch,
    )
    def k(hs_hbm, src3_hbm, dst3_hbm, dis_hbm, z2_hbm, z1_hbm, *rest):
        if do_t:
            (agg_hbm, t_hbm, srcv, dstv, rows, acc, sem,
             val, tacc, sem2) = rest
        else:
            agg_hbm, srcv, dstv, rows, acc, sem = rest
        c = lax.axis_index("c")
        s = lax.axis_index("s")
        w = s * NC + c
        pltpu.sync_copy(z2_hbm.at[pl.ds(s * RPT, RPT)],
                        acc.at[pl.ds(s * RPT, RPT)])
        if do_t:
            pltpu.sync_copy(z1_hbm.at[pl.ds(s * RPT, RPT)],
                            tacc.at[pl.ds(s * RPT, RPT)])
        pltpu.sync_copy(src3_hbm.at[w], srcv)
        pltpu.sync_copy(dst3_hbm.at[w], dstv)
        plsc.subcore_barrier()

        def body(j, carry):
            pltpu.async_copy(hs_hbm.at[srcv.at[j]], rows, sem).wait()
            pltpu.sync_copy(rows, acc.at[dstv.at[j]], add=True)
            if do_t:
                pltpu.async_copy(dis_hbm.at[dstv.at[j]], val, sem2).wait()
                pltpu.sync_copy(val, tacc.at[srcv.at[j]], add=True)
            return carry

        lax.fori_loop(0, NBT, body, 0)
        plsc.subcore_barrier()
        pltpu.sync_copy(acc.at[pl.ds(s * RPT, RPT)],
                        agg_hbm.at[c, pl.ds(s * RPT, RPT)])
        if do_t:
            pltpu.sync_copy(tacc.at[pl.ds(s * RPT, RPT)],
                            t_hbm.at[c, pl.ds(s * RPT, RPT)])

    return k


_sc_agg_t = _make_sc_agg(True)
_sc_agg = _make_sc_agg(False)


RB = 640
GRID = NP // RB


def _tc1_body(x_ref, w1_ref, degp_ref, hs_ref, dis_ref):
    pid = pl.program_id(0)
    deg = degp_ref[0] + degp_ref[1] + 1.0
    rows = pid * RB + lax.broadcasted_iota(jnp.int32, (RB, 1), 0)
    dis = jnp.where(rows < N, lax.rsqrt(deg), 0.0)
    h = jnp.dot(x_ref[...], w1_ref[...], preferred_element_type=jnp.float32)
    hs_ref[...] = dis * h
    dis_ref[...] = dis


def _tc1(x_pad, w1, degp):
    return pl.pallas_call(
        _tc1_body,
        grid=(GRID,),
        in_specs=[
            pl.BlockSpec((RB, D), lambda i: (i, 0)),
            pl.BlockSpec((D, D), lambda i: (0, 0)),
            pl.BlockSpec((NC, RB, 1), lambda i: (0, i, 0)),
        ],
        out_specs=[
            pl.BlockSpec((RB, D), lambda i: (i, 0)),
            pl.BlockSpec((RB, 1), lambda i: (i, 0)),
        ],
        out_shape=[
            jax.ShapeDtypeStruct((NP, D), jnp.float32),
            jax.ShapeDtypeStruct((NP, 1), jnp.float32),
        ],
    )(x_pad, w1, degp)


def _tc2_body(aggp_ref, hs1_ref, dis_ref, b1_ref, w2_ref, hs2_ref):
    dis = dis_ref[...]
    a = aggp_ref[0] + aggp_ref[1] + hs1_ref[...]
    h1 = jnp.maximum(dis * a + b1_ref[...], 0.0)
    hs2_ref[...] = dis * jnp.dot(h1, w2_ref[...],
                                 preferred_element_type=jnp.float32)


def _tc2(aggp, hs1, dis, b1, w2):
    return pl.pallas_call(
        _tc2_body,
        grid=(GRID,),
        in_specs=[
            pl.BlockSpec((NC, RB, D), lambda i: (0, i, 0)),
            pl.BlockSpec((RB, D), lambda i: (i, 0)),
            pl.BlockSpec((RB, 1), lambda i: (i, 0)),
            pl.BlockSpec((1, D), lambda i: (0, 0)),
            pl.BlockSpec((D, D), lambda i: (0, 0)),
        ],
        out_specs=pl.BlockSpec((RB, D), lambda i: (i, 0)),
        out_shape=jax.ShapeDtypeStruct((NP, D), jnp.float32),
    )(aggp, hs1, dis, b1, w2)


def _tc3_body(aggp_ref, hs2_ref, dis_ref, tp_ref, b2_ref, w3_ref, b3_ref,
              m1_ref, mb1_ref, m2_ref, mb2_ref, out_ref, zacc):
    pid = pl.program_id(0)
    dis = dis_ref[...]
    a = aggp_ref[0] + aggp_ref[1] + hs2_ref[...]
    h2 = jnp.maximum(dis * a + b2_ref[...], 0.0)
    cvec = dis * (tp_ref[0] + tp_ref[1] + dis)
    part = jnp.sum(cvec * h2, axis=0, keepdims=True)

    @pl.when(pid == 0)
    def _():
        zacc[...] = jnp.zeros_like(zacc)

    zacc[...] += part

    @pl.when(pid == GRID - 1)
    def _():
        z = zacc[...] * (1.0 / N)
        g = jnp.dot(z, w3_ref[...], preferred_element_type=jnp.float32)
        g = g + b3_ref[...]
        g = jnp.maximum(
            jnp.dot(g, m1_ref[...], preferred_element_type=jnp.float32)
            + mb1_ref[...], 0.0)
        g = jnp.dot(g, m2_ref[...], preferred_element_type=jnp.float32)
        g = g + mb2_ref[...]
        out_ref[...] = g


def _tc3(aggp, hs2, dis, tp, b2, w3, b3, m1, mb1, m2, mb2):
    vec = pl.BlockSpec((1, D), lambda i: (0, 0))
    mat = pl.BlockSpec((D, D), lambda i: (0, 0))
    return pl.pallas_call(
        _tc3_body,
        grid=(GRID,),
        in_specs=[
            pl.BlockSpec((NC, RB, D), lambda i: (0, i, 0)),
            pl.BlockSpec((RB, D), lambda i: (i, 0)),
            pl.BlockSpec((RB, 1), lambda i: (i, 0)),
            pl.BlockSpec((NC, RB, 1), lambda i: (0, i, 0)),
            vec, mat, vec, mat, vec, mat, vec,
        ],
        out_specs=pl.BlockSpec((1, D), lambda i: (0, 0)),
        out_shape=jax.ShapeDtypeStruct((1, D), jnp.float32),
        scratch_shapes=[pltpu.VMEM((1, D), jnp.float32)],
    )(aggp, hs2, dis, tp, b2, w3, b3, m1, mb1, m2, mb2)


def kernel(x, edge_index, W1, b1, W2, b2, W3, b3, M1, mb1, M2, mb2):
    x_pad = jnp.pad(x, ((0, NP - N), (0, 0)))
    pad = jnp.full((EP - E,), NP - 1, dtype=jnp.int32)
    src3 = jnp.concatenate([edge_index[0], pad]).reshape(NW, NBT, B)
    dst3 = jnp.concatenate([edge_index[1], pad]).reshape(NW, NBT, B)
    z1 = jnp.zeros((NP,), jnp.float32)
    z2 = jnp.zeros((NP, D), jnp.float32)

    degp = _sc_deg(dst3, z1)
    hs1, dis = _tc1(x_pad, W1, degp.reshape(NC, NP, 1))
    agg1, t = _sc_agg_t(hs1, src3, dst3, dis.reshape(NP), z2, z1)
    hs2 = _tc2(agg1, hs1, dis, b1.reshape(1, D), W2)
    agg2 = _sc_agg(hs2, src3, dst3, dis.reshape(NP), z2, z1)
    g = _tc3(agg2, hs2, dis, t.reshape(NC, NP, 1), b2.reshape(1, D),
             W3, b3.reshape(1, D), M1, mb1.reshape(1, D), M2,
             mb2.reshape(1, D))
    return g


# speedup vs baseline: 1.5771x; 1.0718x over previous
import functools

import jax
import jax.numpy as jnp
from jax import lax
from jax.experimental import pallas as pl
from jax.experimental.pallas import tpu as pltpu
from jax.experimental.pallas import tpu_sc as plsc

N = 10000
D = 128
E = 320000

NC = 2
NS = 16
NW = NC * NS

NP = 10240
RPT = NP // NS

B = 128
NBT = 79
EPW = NBT * B
EP = NW * EPW

_MESH = dict(core_axis_name="c", subcore_axis_name="s", num_cores=NC,
             num_subcores=NS)


def _sc_deg(dst3, z1):
    @functools.partial(
        pl.kernel,
        out_type=jax.ShapeDtypeStruct((NC, NP), jnp.float32),
        mesh=plsc.VectorSubcoreMesh(**_MESH),
        scratch_types=[
            pltpu.VMEM((NBT, B), jnp.int32),
            pltpu.VMEM((B,), jnp.float32),
            pltpu.VMEM_SHARED((NP,), jnp.float32),
        ],
    )
    def k(dst3_hbm, z1_hbm, degp_hbm, dstv, ones_v, dacc):
        c = lax.axis_index("c")
        s = lax.axis_index("s")
        w = s * NC + c
        pltpu.sync_copy(z1_hbm.at[pl.ds(s * RPT, RPT)],
                        dacc.at[pl.ds(s * RPT, RPT)])
        for i in range(B // 16):
            ones_v[pl.ds(i * 16, 16)] = jnp.ones((16,), jnp.float32)
        pltpu.sync_copy(dst3_hbm.at[w], dstv)
        plsc.subcore_barrier()

        def body(j, carry):
            pltpu.sync_copy(ones_v, dacc.at[dstv.at[j]], add=True)
            return carry

        lax.fori_loop(0, NBT, body, 0)
        plsc.subcore_barrier()
        pltpu.sync_copy(dacc.at[pl.ds(s * RPT, RPT)],
                        degp_hbm.at[c, pl.ds(s * RPT, RPT)])

    return k(dst3, z1)


def _make_sc_agg(do_t):
    outs = [jax.ShapeDtypeStruct((NC, NP, D), jnp.float32)]
    scratch = [
        pltpu.VMEM((NBT, B), jnp.int32),
        pltpu.VMEM((NBT, B), jnp.int32),
        pltpu.VMEM((B, D), jnp.float32),
        pltpu.VMEM_SHARED((NP, D), jnp.float32),
        pltpu.SemaphoreType.DMA,
    ]
    if do_t:
        outs.append(jax.ShapeDtypeStruct((NC, NP), jnp.float32))
        scratch += [
            pltpu.VMEM((B,), jnp.float32),
            pltpu.VMEM_SHARED((NP,), jnp.float32),
            pltpu.SemaphoreType.DMA,
        ]

    @functools.partial(
        pl.kernel,
        out_type=tuple(outs) if do_t else outs[0],
        mesh=plsc.VectorSubcoreMesh(**_MESH),
        scratch_types=scratch,
    )
    def k(hs_hbm, src3_hbm, dst3_hbm, dis_hbm, z2_hbm, z1_hbm, *rest):
        if do_t:
            (agg_hbm, t_hbm, srcv, dstv, rows, acc, sem,
             val, tacc, sem2) = rest
        else:
            agg_hbm, srcv, dstv, rows, acc, sem = rest
        c = lax.axis_index("c")
        s = lax.axis_index("s")
        w = s * NC + c
        pltpu.sync_copy(z2_hbm.at[pl.ds(s * RPT, RPT)],
                        acc.at[pl.ds(s * RPT, RPT)])
        if do_t:
            pltpu.sync_copy(z1_hbm.at[pl.ds(s * RPT, RPT)],
                            tacc.at[pl.ds(s * RPT, RPT)])
        pltpu.sync_copy(src3_hbm.at[w], srcv)
        pltpu.sync_copy(dst3_hbm.at[w], dstv)
        plsc.subcore_barrier()

        def body(j, carry):
            h = pltpu.async_copy(hs_hbm.at[srcv.at[j]], rows, sem)
            if do_t:
                hv = pltpu.async_copy(dis_hbm.at[dstv.at[j]], val, sem2)
            h.wait()
            pltpu.sync_copy(rows, acc.at[dstv.at[j]], add=True)
            if do_t:
                hv.wait()
                pltpu.sync_copy(val, tacc.at[srcv.at[j]], add=True)
            return carry

        lax.fori_loop(0, NBT, body, 0)
        plsc.subcore_barrier()
        pltpu.sync_copy(acc.at[pl.ds(s * RPT, RPT)],
                        agg_hbm.at[c, pl.ds(s * RPT, RPT)])
        if do_t:
            pltpu.sync_copy(tacc.at[pl.ds(s * RPT, RPT)],
                            t_hbm.at[c, pl.ds(s * RPT, RPT)])

    return k


_sc_agg_t = _make_sc_agg(True)
_sc_agg = _make_sc_agg(False)


RB = 640
GRID = NP // RB


def _tc1_body(x_ref, w1_ref, degp_ref, hs_ref, dis_ref):
    pid = pl.program_id(0)
    deg = degp_ref[0] + degp_ref[1] + 1.0
    rows = pid * RB + lax.broadcasted_iota(jnp.int32, (RB, 1), 0)
    dis = jnp.where(rows < N, lax.rsqrt(deg), 0.0)
    h = jnp.dot(x_ref[...], w1_ref[...], preferred_element_type=jnp.float32)
    hs_ref[...] = dis * h
    dis_ref[...] = dis


def _tc1(x_pad, w1, degp):
    return pl.pallas_call(
        _tc1_body,
        grid=(GRID,),
        in_specs=[
            pl.BlockSpec((RB, D), lambda i: (i, 0)),
            pl.BlockSpec((D, D), lambda i: (0, 0)),
            pl.BlockSpec((NC, RB, 1), lambda i: (0, i, 0)),
        ],
        out_specs=[
            pl.BlockSpec((RB, D), lambda i: (i, 0)),
            pl.BlockSpec((RB, 1), lambda i: (i, 0)),
        ],
        out_shape=[
            jax.ShapeDtypeStruct((NP, D), jnp.float32),
            jax.ShapeDtypeStruct((NP, 1), jnp.float32),
        ],
    )(x_pad, w1, degp)


def _tc2_body(aggp_ref, hs1_ref, dis_ref, b1_ref, w2_ref, hs2_ref):
    dis = dis_ref[...]
    a = aggp_ref[0] + aggp_ref[1] + hs1_ref[...]
    h1 = jnp.maximum(dis * a + b1_ref[...], 0.0)
    hs2_ref[...] = dis * jnp.dot(h1, w2_ref[...],
                                 preferred_element_type=jnp.float32)


def _tc2(aggp, hs1, dis, b1, w2):
    return pl.pallas_call(
        _tc2_body,
        grid=(GRID,),
        in_specs=[
            pl.BlockSpec((NC, RB, D), lambda i: (0, i, 0)),
            pl.BlockSpec((RB, D), lambda i: (i, 0)),
            pl.BlockSpec((RB, 1), lambda i: (i, 0)),
            pl.BlockSpec((1, D), lambda i: (0, 0)),
            pl.BlockSpec((D, D), lambda i: (0, 0)),
        ],
        out_specs=pl.BlockSpec((RB, D), lambda i: (i, 0)),
        out_shape=jax.ShapeDtypeStruct((NP, D), jnp.float32),
    )(aggp, hs1, dis, b1, w2)


def _tc3_body(aggp_ref, hs2_ref, dis_ref, tp_ref, b2_ref, w3_ref, b3_ref,
              m1_ref, mb1_ref, m2_ref, mb2_ref, out_ref, zacc):
    pid = pl.program_id(0)
    dis = dis_ref[...]
    a = aggp_ref[0] + aggp_ref[1] + hs2_ref[...]
    h2 = jnp.maximum(dis * a + b2_ref[...], 0.0)
    cvec = dis * (tp_ref[0] + tp_ref[1] + dis)
    part = jnp.sum(cvec * h2, axis=0, keepdims=True)

    @pl.when(pid == 0)
    def _():
        zacc[...] = jnp.zeros_like(zacc)

    zacc[...] += part

    @pl.when(pid == GRID - 1)
    def _():
        z = zacc[...] * (1.0 / N)
        g = jnp.dot(z, w3_ref[...], preferred_element_type=jnp.float32)
        g = g + b3_ref[...]
        g = jnp.maximum(
            jnp.dot(g, m1_ref[...], preferred_element_type=jnp.float32)
            + mb1_ref[...], 0.0)
        g = jnp.dot(g, m2_ref[...], preferred_element_type=jnp.float32)
        g = g + mb2_ref[...]
        out_ref[...] = g


def _tc3(aggp, hs2, dis, tp, b2, w3, b3, m1, mb1, m2, mb2):
    vec = pl.BlockSpec((1, D), lambda i: (0, 0))
    mat = pl.BlockSpec((D, D), lambda i: (0, 0))
    return pl.pallas_call(
        _tc3_body,
        grid=(GRID,),
        in_specs=[
            pl.BlockSpec((NC, RB, D), lambda i: (0, i, 0)),
            pl.BlockSpec((RB, D), lambda i: (i, 0)),
            pl.BlockSpec((RB, 1), lambda i: (i, 0)),
            pl.BlockSpec((NC, RB, 1), lambda i: (0, i, 0)),
            vec, mat, vec, mat, vec, mat, vec,
        ],
        out_specs=pl.BlockSpec((1, D), lambda i: (0, 0)),
        out_shape=jax.ShapeDtypeStruct((1, D), jnp.float32),
        scratch_shapes=[pltpu.VMEM((1, D), jnp.float32)],
    )(aggp, hs2, dis, tp, b2, w3, b3, m1, mb1, m2, mb2)


def kernel(x, edge_index, W1, b1, W2, b2, W3, b3, M1, mb1, M2, mb2):
    x_pad = jnp.pad(x, ((0, NP - N), (0, 0)))
    pad = jnp.full((EP - E,), NP - 1, dtype=jnp.int32)
    src3 = jnp.concatenate([edge_index[0], pad]).reshape(NW, NBT, B)
    dst3 = jnp.concatenate([edge_index[1], pad]).reshape(NW, NBT, B)
    z1 = jnp.zeros((NP,), jnp.float32)
    z2 = jnp.zeros((NP, D), jnp.float32)

    degp = _sc_deg(dst3, z1)
    hs1, dis = _tc1(x_pad, W1, degp.reshape(NC, NP, 1))
    agg1, t = _sc_agg_t(hs1, src3, dst3, dis.reshape(NP), z2, z1)
    hs2 = _tc2(agg1, hs1, dis, b1.reshape(1, D), W2)
    agg2 = _sc_agg(hs2, src3, dst3, dis.reshape(NP), z2, z1)
    g = _tc3(agg2, hs2, dis, t.reshape(NC, NP, 1), b2.reshape(1, D),
             W3, b3.reshape(1, D), M1, mb1.reshape(1, D), M2,
             mb2.reshape(1, D))
    return g


# speedup vs baseline: 2.3170x; 1.4691x over previous
import functools

import jax
import jax.numpy as jnp
from jax import lax
from jax.experimental import pallas as pl
from jax.experimental.pallas import tpu as pltpu
from jax.experimental.pallas import tpu_sc as plsc

N = 10000
D = 128
E = 320000

NC = 2
NS = 16
NW = NC * NS

NP = 10240
RPT = NP // NS

B = 128
NBT0 = 97
NBT1 = 60
EP = NS * (NBT0 + NBT1) * B

_MESH = dict(core_axis_name="c", subcore_axis_name="s", num_cores=NC,
             num_subcores=NS)


def _sc_deg(dstA, dstB, z1):
    @functools.partial(
        pl.kernel,
        out_type=jax.ShapeDtypeStruct((NC, NP), jnp.float32),
        mesh=plsc.VectorSubcoreMesh(**_MESH),
        scratch_types=[
            pltpu.VMEM((NBT0, B), jnp.int32),
            pltpu.VMEM((B,), jnp.float32),
            pltpu.VMEM_SHARED((NP,), jnp.float32),
        ],
    )
    def k(dstA_hbm, dstB_hbm, z1_hbm, degp_hbm, dstv, ones_v, dacc):
        c = lax.axis_index("c")
        s = lax.axis_index("s")
        nbt = jnp.where(c == 0, NBT0, NBT1)
        pltpu.sync_copy(z1_hbm.at[pl.ds(s * RPT, RPT)],
                        dacc.at[pl.ds(s * RPT, RPT)])
        for i in range(B // 16):
            ones_v[pl.ds(i * 16, 16)] = jnp.ones((16,), jnp.float32)

        @pl.when(c == 0)
        def _():
            pltpu.sync_copy(dstA_hbm.at[s], dstv.at[pl.ds(0, NBT0)])

        @pl.when(c == 1)
        def _():
            pltpu.sync_copy(dstB_hbm.at[s], dstv.at[pl.ds(0, NBT1)])

        plsc.subcore_barrier()

        def body(j, carry):
            pltpu.sync_copy(ones_v, dacc.at[dstv.at[j]], add=True)
            return carry

        lax.fori_loop(0, nbt, body, 0)
        plsc.subcore_barrier()
        pltpu.sync_copy(dacc.at[pl.ds(s * RPT, RPT)],
                        degp_hbm.at[c, pl.ds(s * RPT, RPT)])

    return k(dstA, dstB, z1)


def _make_sc_agg(do_t):
    outs = [jax.ShapeDtypeStruct((NC, NP, D), jnp.float32)]
    scratch = [
        pltpu.VMEM((NBT0, B), jnp.int32),
        pltpu.VMEM((NBT0, B), jnp.int32),
        pltpu.VMEM((B, D), jnp.float32),
        pltpu.VMEM_SHARED((NP, D), jnp.float32),
        pltpu.SemaphoreType.DMA,
    ]
    if do_t:
        outs.append(jax.ShapeDtypeStruct((NC, NP), jnp.float32))
        scratch += [
            pltpu.VMEM((B,), jnp.float32),
            pltpu.VMEM_SHARED((NP,), jnp.float32),
            pltpu.SemaphoreType.DMA,
        ]

    @functools.partial(
        pl.kernel,
        out_type=tuple(outs) if do_t else outs[0],
        mesh=plsc.VectorSubcoreMesh(**_MESH),
        scratch_types=scratch,
    )
    def k(hs_hbm, srcA_hbm, dstA_hbm, srcB_hbm, dstB_hbm, dis_hbm,
          z2_hbm, z1_hbm, *rest):
        if do_t:
            (agg_hbm, t_hbm, srcv, dstv, rows, acc, sem,
             val, tacc, sem2) = rest
        else:
            agg_hbm, srcv, dstv, rows, acc, sem = rest
        c = lax.axis_index("c")
        s = lax.axis_index("s")
        nbt = jnp.where(c == 0, NBT0, NBT1)
        pltpu.sync_copy(z2_hbm.at[pl.ds(s * RPT, RPT)],
                        acc.at[pl.ds(s * RPT, RPT)])
        if do_t:
            pltpu.sync_copy(z1_hbm.at[pl.ds(s * RPT, RPT)],
                            tacc.at[pl.ds(s * RPT, RPT)])

        @pl.when(c == 0)
        def _():
            pltpu.sync_copy(srcA_hbm.at[s], srcv.at[pl.ds(0, NBT0)])
            pltpu.sync_copy(dstA_hbm.at[s], dstv.at[pl.ds(0, NBT0)])

        @pl.when(c == 1)
        def _():
            pltpu.sync_copy(srcB_hbm.at[s], srcv.at[pl.ds(0, NBT1)])
            pltpu.sync_copy(dstB_hbm.at[s], dstv.at[pl.ds(0, NBT1)])

        plsc.subcore_barrier()

        def body(j, carry):
            h = pltpu.async_copy(hs_hbm.at[srcv.at[j]], rows, sem)
            if do_t:
                hv = pltpu.async_copy(dis_hbm.at[dstv.at[j]], val, sem2)
            h.wait()
            pltpu.sync_copy(rows, acc.at[dstv.at[j]], add=True)
            if do_t:
                hv.wait()
                pltpu.sync_copy(val, tacc.at[srcv.at[j]], add=True)
            return carry

        lax.fori_loop(0, nbt, body, 0)
        plsc.subcore_barrier()
        pltpu.sync_copy(acc.at[pl.ds(s * RPT, RPT)],
                        agg_hbm.at[c, pl.ds(s * RPT, RPT)])
        if do_t:
            pltpu.sync_copy(tacc.at[pl.ds(s * RPT, RPT)],
                            t_hbm.at[c, pl.ds(s * RPT, RPT)])

    return k


_sc_agg_t = _make_sc_agg(True)
_sc_agg = _make_sc_agg(False)


RB = 640
GRID = NP // RB


def _tc1_body(x_ref, w1_ref, degp_ref, hs_ref, dis_ref):
    pid = pl.program_id(0)
    deg = degp_ref[0] + degp_ref[1] + 1.0
    rows = pid * RB + lax.broadcasted_iota(jnp.int32, (RB, 1), 0)
    dis = jnp.where(rows < N, lax.rsqrt(deg), 0.0)
    h = jnp.dot(x_ref[...], w1_ref[...], preferred_element_type=jnp.float32)
    hs_ref[...] = dis * h
    dis_ref[...] = dis


def _tc1(x_pad, w1, degp):
    return pl.pallas_call(
        _tc1_body,
        grid=(GRID,),
        in_specs=[
            pl.BlockSpec((RB, D), lambda i: (i, 0)),
            pl.BlockSpec((D, D), lambda i: (0, 0)),
            pl.BlockSpec((NC, RB, 1), lambda i: (0, i, 0)),
        ],
        out_specs=[
            pl.BlockSpec((RB, D), lambda i: (i, 0)),
            pl.BlockSpec((RB, 1), lambda i: (i, 0)),
        ],
        out_shape=[
            jax.ShapeDtypeStruct((NP, D), jnp.float32),
            jax.ShapeDtypeStruct((NP, 1), jnp.float32),
        ],
    )(x_pad, w1, degp)


def _tc2_body(aggp_ref, hs1_ref, dis_ref, b1_ref, w2_ref, hs2_ref):
    dis = dis_ref[...]
    a = aggp_ref[0] + aggp_ref[1] + hs1_ref[...]
    h1 = jnp.maximum(dis * a + b1_ref[...], 0.0)
    hs2_ref[...] = dis * jnp.dot(h1, w2_ref[...],
                                 preferred_element_type=jnp.float32)


def _tc2(aggp, hs1, dis, b1, w2):
    return pl.pallas_call(
        _tc2_body,
        grid=(GRID,),
        in_specs=[
            pl.BlockSpec((NC, RB, D), lambda i: (0, i, 0)),
            pl.BlockSpec((RB, D), lambda i: (i, 0)),
            pl.BlockSpec((RB, 1), lambda i: (i, 0)),
            pl.BlockSpec((1, D), lambda i: (0, 0)),
            pl.BlockSpec((D, D), lambda i: (0, 0)),
        ],
        out_specs=pl.BlockSpec((RB, D), lambda i: (i, 0)),
        out_shape=jax.ShapeDtypeStruct((NP, D), jnp.float32),
    )(aggp, hs1, dis, b1, w2)


def _tc3_body(aggp_ref, hs2_ref, dis_ref, tp_ref, b2_ref, w3_ref, b3_ref,
              m1_ref, mb1_ref, m2_ref, mb2_ref, out_ref, zacc):
    pid = pl.program_id(0)
    dis = dis_ref[...]
    a = aggp_ref[0] + aggp_ref[1] + hs2_ref[...]
    h2 = jnp.maximum(dis * a + b2_ref[...], 0.0)
    cvec = dis * (tp_ref[0] + tp_ref[1] + dis)
    part = jnp.sum(cvec * h2, axis=0, keepdims=True)

    @pl.when(pid == 0)
    def _():
        zacc[...] = jnp.zeros_like(zacc)

    zacc[...] += part

    @pl.when(pid == GRID - 1)
    def _():
        z = zacc[...] * (1.0 / N)
        g = jnp.dot(z, w3_ref[...], preferred_element_type=jnp.float32)
        g = g + b3_ref[...]
        g = jnp.maximum(
            jnp.dot(g, m1_ref[...], preferred_element_type=jnp.float32)
            + mb1_ref[...], 0.0)
        g = jnp.dot(g, m2_ref[...], preferred_element_type=jnp.float32)
        g = g + mb2_ref[...]
        out_ref[...] = g


def _tc3(aggp, hs2, dis, tp, b2, w3, b3, m1, mb1, m2, mb2):
    vec = pl.BlockSpec((1, D), lambda i: (0, 0))
    mat = pl.BlockSpec((D, D), lambda i: (0, 0))
    return pl.pallas_call(
        _tc3_body,
        grid=(GRID,),
        in_specs=[
            pl.BlockSpec((NC, RB, D), lambda i: (0, i, 0)),
            pl.BlockSpec((RB, D), lambda i: (i, 0)),
            pl.BlockSpec((RB, 1), lambda i: (i, 0)),
            pl.BlockSpec((NC, RB, 1), lambda i: (0, i, 0)),
            vec, mat, vec, mat, vec, mat, vec,
        ],
        out_specs=pl.BlockSpec((1, D), lambda i: (0, 0)),
        out_shape=jax.ShapeDtypeStruct((1, D), jnp.float32),
        scratch_shapes=[pltpu.VMEM((1, D), jnp.float32)],
    )(aggp, hs2, dis, tp, b2, w3, b3, m1, mb1, m2, mb2)


def kernel(x, edge_index, W1, b1, W2, b2, W3, b3, M1, mb1, M2, mb2):
    x_pad = jnp.pad(x, ((0, NP - N), (0, 0)))
    pad = jnp.full((EP - E,), NP - 1, dtype=jnp.int32)
    EA = NS * NBT0 * B
    src_f = jnp.concatenate([edge_index[0], pad])
    dst_f = jnp.concatenate([edge_index[1], pad])
    srcA = src_f[:EA].reshape(NS, NBT0, B)
    dstA = dst_f[:EA].reshape(NS, NBT0, B)
    srcB = src_f[EA:].reshape(NS, NBT1, B)
    dstB = dst_f[EA:].reshape(NS, NBT1, B)
    z1 = jnp.zeros((NP,), jnp.float32)
    z2 = jnp.zeros((NP, D), jnp.float32)

    degp = _sc_deg(dstA, dstB, z1)
    hs1, dis = _tc1(x_pad, W1, degp.reshape(NC, NP, 1))
    dis1 = dis.reshape(NP)
    agg1, t = _sc_agg_t(hs1, srcA, dstA, srcB, dstB, dis1, z2, z1)
    hs2 = _tc2(agg1, hs1, dis, b1.reshape(1, D), W2)
    agg2 = _sc_agg(hs2, srcA, dstA, srcB, dstB, dis1, z2, z1)
    g = _tc3(agg2, hs2, dis, t.reshape(NC, NP, 1), b2.reshape(1, D),
             W3, b3.reshape(1, D), M1, mb1.reshape(1, D), M2,
             mb2.reshape(1, D))
    return g
